# baseline (speedup 1.0000x reference)
import functools

import jax
import jax.numpy as jnp
from jax import lax
from jax.experimental import pallas as pl
from jax.experimental.pallas import tpu as pltpu
from jax.experimental.pallas import tpu_sc as plsc

N = 8388608
K_SEL = int(0.01 * N)
NC, NS, L = 2, 16, 16
NW = NC * NS
PER_TILE = N // NW
CHUNK = 16384

B0 = 32768
B1 = 65536
SH0 = 16

_mesh = plsc.VectorSubcoreMesh(
    core_axis_name="c", subcore_axis_name="s", num_cores=NC, num_subcores=NS)
_sc_params = pltpu.CompilerParams(needs_layout_passes=False)


def _wid():
    return lax.axis_index("s") * NC + lax.axis_index("c")


def _abs_bits(x):
    return lax.bitcast_convert_type(x, jnp.int32) & jnp.int32(0x7FFFFFFF)


def _zero_ref(ref, nwords):
    z = jnp.zeros((L,), jnp.int32)

    @plsc.parallel_loop(0, nwords // L, 1, unroll=8)
    def _(i):
        ref[pl.ds(i * L, L)] = z


def _hist_pass(tv_hbm, buf0, buf1, sem0, sem1, hist, bucket_fn):
    ones = jnp.ones((L,), jnp.int32)
    base = _wid() * PER_TILE
    nchunks = PER_TILE // CHUNK

    def src(ci):
        return tv_hbm.at[pl.ds(base + ci * CHUNK, CHUNK)]

    pltpu.async_copy(src(0), buf0, sem0)
    pltpu.async_copy(src(1), buf1, sem1)

    def process(buf):
        @plsc.parallel_loop(0, CHUNK // L, 1, unroll=16)
        def _(j):
            x = buf[pl.ds(j * L, L)]
            bits = _abs_bits(x)
            bkt, mask = bucket_fn(bits)
            plsc.addupdate_scatter(hist, [bkt], ones, mask=mask)

    def outer(t, carry):
        ci = t * 2
        for b, (buf, sem) in enumerate(((buf0, sem0), (buf1, sem1))):
            pltpu.make_async_copy(src(ci + b), buf, sem).wait()
            process(buf)

            @pl.when(ci + b + 2 < nchunks)
            def _():
                pltpu.async_copy(src(ci + b + 2), buf, sem)

        return carry

    lax.fori_loop(0, nchunks // 2, outer, 0)


def _load_sel(sel_hbm, selv, nvals):
    pltpu.sync_copy(sel_hbm, selv)
    lane = lax.iota(jnp.int32, L)
    v = selv[...]
    big_neg = jnp.int32(-(2 ** 31))
    return [jnp.max(jnp.where(lane == i, v, big_neg)) for i in range(nvals)]


@functools.partial(
    pl.kernel,
    out_type=jax.ShapeDtypeStruct((NW * B0,), jnp.int32),
    mesh=_mesh,
    compiler_params=_sc_params,
    scratch_types=[
        pltpu.VMEM((CHUNK,), jnp.float32),
        pltpu.VMEM((CHUNK,), jnp.float32),
        pltpu.SemaphoreType.DMA,
        pltpu.SemaphoreType.DMA,
        pltpu.VMEM((B0,), jnp.int32),
        pltpu.SemaphoreType.DMA,
    ],
)
def _k_hist0(tv_hbm, zz_hbm, h0_hbm, buf0, buf1, sem0, sem1, hist, zsem):
    pltpu.async_copy(zz_hbm.at[pl.ds(0, B0)], hist, zsem)
    pltpu.make_async_copy(zz_hbm.at[pl.ds(0, B0)], hist, zsem).wait()
    _hist_pass(tv_hbm, buf0, buf1, sem0, sem1, hist,
               lambda bits: (bits >> SH0, None))
    pltpu.sync_copy(hist, h0_hbm.at[pl.ds(_wid() * B0, B0)])


@functools.partial(
    pl.kernel,
    out_type=jax.ShapeDtypeStruct((NW * B1,), jnp.int32),
    mesh=_mesh,
    compiler_params=_sc_params,
    scratch_types=[
        pltpu.VMEM((CHUNK,), jnp.float32),
        pltpu.VMEM((CHUNK,), jnp.float32),
        pltpu.SemaphoreType.DMA,
        pltpu.SemaphoreType.DMA,
        pltpu.VMEM((B1,), jnp.int32),
        pltpu.VMEM((L,), jnp.int32),
        pltpu.SemaphoreType.DMA,
    ],
)
def _k_hist1(tv_hbm, sel0_hbm, zz_hbm, h1_hbm, buf0, buf1, sem0, sem1,
             hist, selv, zsem):
    pltpu.async_copy(zz_hbm, hist, zsem)
    b0, = _load_sel(sel0_hbm, selv, 1)
    pltpu.make_async_copy(zz_hbm, hist, zsem).wait()

    def bucket_fn(bits):
        mask = (bits >> SH0) == b0
        bkt = bits & jnp.int32(B1 - 1)
        return bkt, mask

    _hist_pass(tv_hbm, buf0, buf1, sem0, sem1, hist, bucket_fn)
    pltpu.sync_copy(hist, h1_hbm.at[pl.ds(_wid() * B1, B1)])


def _bsearch(g, flat_idx, kk, nbits):
    def bs(i, lohi):
        lo, hi = lohi
        mid = (lo + hi + 1) // 2
        cnt = jnp.sum(jnp.where(flat_idx >= mid, g, 0))
        ge = cnt >= kk
        return (jnp.where(ge, mid, lo), jnp.where(ge, hi, mid - 1))

    lo, _ = lax.fori_loop(0, nbits, bs, (jnp.int32(0), jnp.int32(2 ** nbits - 1)))
    cnt_gt = jnp.sum(jnp.where(flat_idx > lo, g, 0))
    return lo, cnt_gt


def _sel0_body(h_ref, sel_ref):
    g = jnp.sum(h_ref[...], axis=0)
    rows = B0 // 128
    flat = (lax.broadcasted_iota(jnp.int32, (rows, 128), 0) * 128
            + lax.broadcasted_iota(jnp.int32, (rows, 128), 1))
    b0, cnt_gt = _bsearch(g, flat, jnp.int32(K_SEL), 15)
    sel_ref[0] = b0
    sel_ref[1] = jnp.int32(K_SEL) - cnt_gt
    for i in range(2, L):
        sel_ref[i] = jnp.int32(0)


def _tc_sel0(h0):
    return pl.pallas_call(
        _sel0_body,
        in_specs=[pl.BlockSpec((NW, B0 // 128, 128), lambda: (0, 0, 0))],
        out_specs=pl.BlockSpec(memory_space=pltpu.SMEM),
        out_shape=jax.ShapeDtypeStruct((L,), jnp.int32),
    )(h0)


ROWS, COLS = 65536, 128
BLK_ROWS = 4096


def _tc_ew_body(sel0_ref, h1_hbm, tv_ref, pre_ref, out_ref, hbuf, hsem, thr_ref):
    pid = pl.program_id(0)

    @pl.when(pid == 0)
    def _():
        pltpu.make_async_copy(h1_hbm, hbuf, hsem).start()
        pltpu.make_async_copy(h1_hbm, hbuf, hsem).wait()
        g = jnp.sum(hbuf[...], axis=0)
        rows = B1 // 128
        flat = (lax.broadcasted_iota(jnp.int32, (rows, 128), 0) * 128
                + lax.broadcasted_iota(jnp.int32, (rows, 128), 1))
        b1, _ = _bsearch(g, flat, sel0_ref[1], 16)
        thr_ref[0] = (sel0_ref[0] << SH0) | b1

    tb = thr_ref[0]
    tv = tv_ref[...]
    bits = lax.bitcast_convert_type(tv, jnp.int32) & jnp.int32(0x7FFFFFFF)
    bp = jnp.where(bits > tb, jnp.float32(5.0), jnp.float32(-5.0))
    frac = jax.nn.sigmoid(bp)
    out_ref[...] = pre_ref[...] + frac * tv


def _tc_finish(sel0, h1, tv2d, pre2d):
    return pl.pallas_call(
        _tc_ew_body,
        grid=(ROWS // BLK_ROWS,),
        in_specs=[
            pl.BlockSpec(memory_space=pltpu.SMEM),
            pl.BlockSpec(memory_space=pl.ANY),
            pl.BlockSpec((BLK_ROWS, COLS), lambda i: (i, 0)),
            pl.BlockSpec((BLK_ROWS, COLS), lambda i: (i, 0)),
        ],
        out_specs=pl.BlockSpec((BLK_ROWS, COLS), lambda i: (i, 0)),
        out_shape=jax.ShapeDtypeStruct((ROWS, COLS), jnp.float32),
        scratch_shapes=[
            pltpu.VMEM((NW, B1 // 128, 128), jnp.int32),
            pltpu.SemaphoreType.DMA,
            pltpu.SMEM((1,), jnp.int32),
        ],
    )(sel0, h1, tv2d, pre2d)


@jax.jit
def kernel(task_vector, pretensor):
    tv = task_vector.reshape(-1)
    zz = jnp.zeros((B1,), jnp.int32)
    h0 = _k_hist0(tv, zz)
    sel0 = _tc_sel0(h0.reshape(NW, B0 // 128, 128))
    h1 = _k_hist1(tv, sel0, zz)
    out2d = _tc_finish(sel0, h1.reshape(NW, B1 // 128, 128),
                       tv.reshape(ROWS, COLS), pretensor.reshape(ROWS, COLS))
    return out2d.reshape(task_vector.shape)
